# baseline (speedup 1.0000x reference)
import functools

import jax
import jax.numpy as jnp
from jax import lax
from jax.experimental import pallas as pl
from jax.experimental.pallas import tpu as pltpu
from jax.experimental.pallas import tpu_sc as plsc

N_ROWS = 1_000_000
D = 32
N_IDX = 16_384
N_TILES = 16
B = N_IDX // N_TILES
CHUNK = 128
N_CHUNKS = B // CHUNK


_COPY_ROWS = 250_000
_COPY_BLOCK = 10_000


def _copy_body(x_ref, o_ref):
  o_ref[...] = x_ref[...]


_tc_copy = pl.pallas_call(
    _copy_body,
    out_shape=jax.ShapeDtypeStruct((_COPY_ROWS, 128), jnp.float32),
    grid=(_COPY_ROWS // _COPY_BLOCK,),
    in_specs=[pl.BlockSpec((_COPY_BLOCK, 128), lambda i: (i, 0))],
    out_specs=pl.BlockSpec((_COPY_BLOCK, 128), lambda i: (i, 0)),
    compiler_params=pltpu.CompilerParams(
        dimension_semantics=("arbitrary",)),
)


_mesh = plsc.VectorSubcoreMesh(
    core_axis_name="c", subcore_axis_name="s", num_cores=1)


def _sc_body(out_hbm, x_hbm, idx_hbm, val_hbm, zero_hbm, slot_hbm,
             own_sh, acc_sh, idx_v, w_v, slot_v, a_v, b_v):
  wid = lax.axis_index("s")
  base = wid * B

  pltpu.sync_copy(idx_hbm.at[wid], idx_v)
  pltpu.sync_copy(slot_hbm.at[wid], slot_v)
  pltpu.sync_copy(zero_hbm.at[pl.ds(base, B)], acc_sh.at[pl.ds(base, B)])
  for j in range(N_CHUNKS):
    pltpu.sync_copy(slot_v.at[j], own_sh.at[idx_v.at[j]])

  plsc.subcore_barrier()

  for j in range(N_CHUNKS):
    pltpu.sync_copy(own_sh.at[idx_v.at[j]], w_v.at[j])
    pltpu.sync_copy(val_hbm.at[wid, pl.ds(j * CHUNK, CHUNK)], a_v)
    pltpu.sync_copy(a_v, acc_sh.at[w_v.at[j]], add=True)

  plsc.subcore_barrier()

  for j in range(N_CHUNKS):
    pltpu.sync_copy(acc_sh.at[w_v.at[j]], a_v)
    pltpu.sync_copy(x_hbm.at[idx_v.at[j]], b_v)

    @pl.loop(0, CHUNK)
    def _add_rows(r):
      for c in range(D // 16):
        sl = pl.ds(c * 16, 16)
        b_v[r, sl] = b_v[r, sl] + a_v[r, sl]

    pltpu.sync_copy(b_v, out_hbm.at[idx_v.at[j]])


_sc_rmw = pl.kernel(
    _sc_body,
    out_type=(),
    mesh=_mesh,
    scratch_types=[
        pltpu.VMEM_SHARED((N_ROWS,), jnp.int32),
        pltpu.VMEM_SHARED((N_IDX, D), jnp.float32),
        pltpu.VMEM((N_CHUNKS, CHUNK), jnp.int32),
        pltpu.VMEM((N_CHUNKS, CHUNK), jnp.int32),
        pltpu.VMEM((N_CHUNKS, CHUNK), jnp.int32),
        pltpu.VMEM((CHUNK, D), jnp.float32),
        pltpu.VMEM((CHUNK, D), jnp.float32),
    ],
    compiler_params=pltpu.CompilerParams(use_tc_tiling_on_sc=False),
)


@functools.partial(jax.jit, static_argnums=())
def kernel(x, indices, values):
  idx = indices.astype(jnp.int32).reshape(N_TILES, N_CHUNKS, CHUNK)
  vals = values.astype(jnp.float32).reshape(N_TILES, B, D)
  zeros = jnp.zeros((N_IDX, D), jnp.float32)
  slots = jnp.arange(N_IDX, dtype=jnp.int32).reshape(
      N_TILES, N_CHUNKS, CHUNK)

  out0 = _tc_copy(x.reshape(_COPY_ROWS, 128)).reshape(N_ROWS, D)
  out_ref = jax.new_ref(out0)
  _sc_rmw(out_ref, x, idx, vals, zeros, slots)
  return jax.freeze(out_ref)


# speedup vs baseline: 1.4027x; 1.4027x over previous
import jax
import jax.numpy as jnp
from jax import lax
from jax.experimental import pallas as pl
from jax.experimental.pallas import tpu as pltpu
from jax.experimental.pallas import tpu_sc as plsc

N_ROWS = 1_000_000
D = 32
N_IDX = 16_384
N_TILES = 16
B = N_IDX // N_TILES
CHUNK = 128
N_CHUNKS = B // CHUNK

_mesh = plsc.VectorSubcoreMesh(
    core_axis_name="c", subcore_axis_name="s", num_cores=1)


def _sc_body(out_hbm, idx_hbm, val_hbm, zero_hbm, slot_hbm,
             own_sh, acc_sh, idx_v, w_v, slot_v, wd_v, a_v):
  wid = lax.axis_index("s")
  base = wid * B

  pltpu.sync_copy(idx_hbm.at[wid], idx_v)
  pltpu.sync_copy(slot_hbm.at[wid], slot_v)
  pltpu.sync_copy(zero_hbm.at[pl.ds(base, B)], acc_sh.at[pl.ds(base, B)])
  for j in range(N_CHUNKS):
    pltpu.sync_copy(slot_v.at[j], own_sh.at[idx_v.at[j]])

  plsc.subcore_barrier()

  for j in range(N_CHUNKS):
    pltpu.sync_copy(own_sh.at[idx_v.at[j]], w_v.at[j])
  for t in range(N_CHUNKS * CHUNK // 16):
    r, c = t // 8, (t % 8) * 16
    sl = pl.ds(c, 16)
    w16 = w_v[r, sl]
    wd_v[r, sl] = jnp.where(w16 == slot_v[r, sl], w16, N_IDX)

  for j in range(N_CHUNKS):
    pltpu.sync_copy(val_hbm.at[wid, pl.ds(j * CHUNK, CHUNK)], a_v)
    pltpu.sync_copy(a_v, acc_sh.at[w_v.at[j]], add=True)
    pltpu.sync_copy(out_hbm.at[idx_v.at[j]], a_v)
    pltpu.sync_copy(a_v, acc_sh.at[wd_v.at[j]], add=True)

  plsc.subcore_barrier()

  for j in range(N_CHUNKS):
    pltpu.sync_copy(acc_sh.at[w_v.at[j]], a_v)
    pltpu.sync_copy(a_v, out_hbm.at[idx_v.at[j]])


_sc_rmw = pl.kernel(
    _sc_body,
    out_type=(),
    mesh=_mesh,
    scratch_types=[
        pltpu.VMEM_SHARED((N_ROWS,), jnp.int32),
        pltpu.VMEM_SHARED((N_IDX + 1, D), jnp.float32),
        pltpu.VMEM((N_CHUNKS, CHUNK), jnp.int32),
        pltpu.VMEM((N_CHUNKS, CHUNK), jnp.int32),
        pltpu.VMEM((N_CHUNKS, CHUNK), jnp.int32),
        pltpu.VMEM((N_CHUNKS, CHUNK), jnp.int32),
        pltpu.VMEM((CHUNK, D), jnp.float32),
    ],
    compiler_params=pltpu.CompilerParams(use_tc_tiling_on_sc=False),
)


def kernel(x, indices, values):
  idx = indices.astype(jnp.int32).reshape(N_TILES, N_CHUNKS, CHUNK)
  vals = values.astype(jnp.float32).reshape(N_TILES, B, D)
  zeros = jnp.zeros((N_IDX, D), jnp.float32)
  slots = jnp.arange(N_IDX, dtype=jnp.int32).reshape(
      N_TILES, N_CHUNKS, CHUNK)

  out_ref = jax.new_ref(x)
  _sc_rmw(out_ref, idx, vals, zeros, slots)
  return jax.freeze(out_ref)


# speedup vs baseline: 1.5424x; 1.0996x over previous
import jax
import jax.numpy as jnp
from jax import lax
from jax.experimental import pallas as pl
from jax.experimental.pallas import tpu as pltpu
from jax.experimental.pallas import tpu_sc as plsc

N_ROWS = 1_000_000
D = 32
W = 128
N_IDX = 16_384
N_TILES = 16
B = N_IDX // N_TILES
CHUNK = 128
N_CHUNKS = B // CHUNK

NB = 2048
GRID = (N_ROWS + NB - 1) // NB


def _t_fwd_body(xt_ref, o_ref):
  o_ref[:, pl.ds(0, D)] = xt_ref[...].T


_tc_fwd = pl.pallas_call(
    _t_fwd_body,
    out_shape=jax.ShapeDtypeStruct((N_ROWS, W), jnp.float32),
    grid=(GRID,),
    in_specs=[pl.BlockSpec((D, NB), lambda i: (0, i))],
    out_specs=pl.BlockSpec((NB, W), lambda i: (i, 0)),
    compiler_params=pltpu.CompilerParams(
        dimension_semantics=("arbitrary",)),
)


def _t_bwd_body(big_ref, o_ref):
  o_ref[...] = big_ref[:, pl.ds(0, D)].T


_tc_bwd = pl.pallas_call(
    _t_bwd_body,
    out_shape=jax.ShapeDtypeStruct((D, N_ROWS), jnp.float32),
    grid=(GRID,),
    in_specs=[pl.BlockSpec((NB, W), lambda i: (i, 0))],
    out_specs=pl.BlockSpec((D, NB), lambda i: (0, i)),
    compiler_params=pltpu.CompilerParams(
        dimension_semantics=("arbitrary",)),
)


_mesh = plsc.VectorSubcoreMesh(
    core_axis_name="c", subcore_axis_name="s", num_cores=1)


def _sc_body(out_hbm, idx_hbm, val_hbm, zero_hbm, slot_hbm,
             own_sh, acc_sh, idx_v, w_v, slot_v, wd_v, a_v, a128_v):
  wid = lax.axis_index("s")
  base = wid * B

  pltpu.sync_copy(idx_hbm.at[wid], idx_v)
  pltpu.sync_copy(slot_hbm.at[wid], slot_v)
  pltpu.sync_copy(zero_hbm.at[pl.ds(base, B)], acc_sh.at[pl.ds(base, B)])
  for j in range(N_CHUNKS):
    pltpu.sync_copy(slot_v.at[j], own_sh.at[idx_v.at[j]])

  plsc.subcore_barrier()

  for j in range(N_CHUNKS):
    pltpu.sync_copy(own_sh.at[idx_v.at[j]], w_v.at[j])
  for t in range(N_CHUNKS * CHUNK // 16):
    r, c = t // 8, (t % 8) * 16
    sl = pl.ds(c, 16)
    w16 = w_v[r, sl]
    wd_v[r, sl] = jnp.where(w16 == slot_v[r, sl], w16, N_IDX)

  for j in range(N_CHUNKS):
    pltpu.sync_copy(val_hbm.at[wid, pl.ds(j * CHUNK, CHUNK)], a_v)
    pltpu.sync_copy(a_v, acc_sh.at[w_v.at[j]], add=True)
    pltpu.sync_copy(out_hbm.at[idx_v.at[j]], a128_v)

    @pl.loop(0, CHUNK)
    def _extract(r):
      for c in range(D // 16):
        sl = pl.ds(c * 16, 16)
        a_v[r, sl] = a128_v[r, sl]

    pltpu.sync_copy(a_v, acc_sh.at[wd_v.at[j]], add=True)

  plsc.subcore_barrier()

  for j in range(N_CHUNKS):
    pltpu.sync_copy(acc_sh.at[w_v.at[j]], a_v)

    @pl.loop(0, CHUNK)
    def _inject(r):
      for c in range(D // 16):
        sl = pl.ds(c * 16, 16)
        a128_v[r, sl] = a_v[r, sl]

    pltpu.sync_copy(a128_v, out_hbm.at[idx_v.at[j]])


_sc_rmw = pl.kernel(
    _sc_body,
    out_type=(),
    mesh=_mesh,
    scratch_types=[
        pltpu.VMEM_SHARED((N_ROWS,), jnp.int32),
        pltpu.VMEM_SHARED((N_IDX + 1, D), jnp.float32),
        pltpu.VMEM((N_CHUNKS, CHUNK), jnp.int32),
        pltpu.VMEM((N_CHUNKS, CHUNK), jnp.int32),
        pltpu.VMEM((N_CHUNKS, CHUNK), jnp.int32),
        pltpu.VMEM((N_CHUNKS, CHUNK), jnp.int32),
        pltpu.VMEM((CHUNK, D), jnp.float32),
        pltpu.VMEM((CHUNK, W), jnp.float32),
    ],
    compiler_params=pltpu.CompilerParams(use_tc_tiling_on_sc=False),
)


def kernel(x, indices, values):
  idx = indices.astype(jnp.int32).reshape(N_TILES, N_CHUNKS, CHUNK)
  vals = values.astype(jnp.float32).reshape(N_TILES, B, D)
  zeros = jnp.zeros((N_IDX, D), jnp.float32)
  slots = jnp.arange(N_IDX, dtype=jnp.int32).reshape(
      N_TILES, N_CHUNKS, CHUNK)

  big = _tc_fwd(x.T)
  big_ref = jax.new_ref(big)
  _sc_rmw(big_ref, idx, vals, zeros, slots)
  return _tc_bwd(jax.freeze(big_ref)).T


# speedup vs baseline: 5.3181x; 3.4480x over previous
import jax
import jax.numpy as jnp
from jax import lax
from jax.experimental import pallas as pl
from jax.experimental.pallas import tpu as pltpu
from jax.experimental.pallas import tpu_sc as plsc

N_ROWS = 1_000_000
D = 32
N_IDX = 16_384
WCOLS = 1536
MAIN_COLS = 999_936
N_WIN = MAIN_COLS // WCOLS
N_WORKERS = 32
VCH = 128
SVB = 136

_mesh = plsc.VectorSubcoreMesh(core_axis_name="c", subcore_axis_name="s")


def _sc_body(xt_hbm, sidx_hbm, sval_hbm, wstart_hbm, out_hbm,
             win_v, sidx_v, sval_v, wstart_v):
  nc = 2
  wid = lax.axis_index("s") * nc + lax.axis_index("c")

  pltpu.sync_copy(sidx_hbm, sidx_v.at[pl.ds(0, N_IDX)])
  pltpu.sync_copy(wstart_hbm, wstart_v.at[pl.ds(0, N_WIN + 5)])

  iota16 = lax.iota(jnp.int32, 16)

  @pl.loop(wid, N_WIN, step=N_WORKERS)
  def _window(w):
    col0 = w * WCOLS
    pltpu.sync_copy(xt_hbm.at[:, pl.ds(col0, WCOLS)], win_v)
    bounds = plsc.load_gather(wstart_v, [w + iota16])
    s0 = bounds[0]
    e0 = bounds[1]
    nch = (e0 - s0 + (VCH - 1)) // VCH

    @pl.loop(0, nch)
    def _chunk(k):
      off = s0 + k * VCH
      offc = jnp.minimum((off // 8) * 8, N_IDX - SVB)
      pltpu.sync_copy(sval_hbm.at[pl.ds(offc, SVB)], sval_v)
      lim = jnp.minimum(e0 - off, VCH)

      @pl.loop(0, lim)
      def _pos(t):
        p = off + t
        rsplat = jnp.full((16,), p - offc, jnp.int32)
        cvec = plsc.load_gather(sidx_v, [jnp.full((16,), p, jnp.int32)])
        cvec = cvec - col0
        v0 = plsc.load_gather(sval_v, [rsplat, iota16])
        v1 = plsc.load_gather(sval_v, [rsplat, iota16 + 16])
        plsc.addupdate_scatter(win_v, [iota16, cvec], v0)
        plsc.addupdate_scatter(win_v, [iota16 + 16, cvec], v1)

    pltpu.sync_copy(win_v, out_hbm.at[:, pl.ds(col0, WCOLS)])


_sc_sweep = pl.kernel(
    _sc_body,
    out_type=jax.ShapeDtypeStruct((D, N_ROWS), jnp.float32),
    mesh=_mesh,
    scratch_types=[
        pltpu.VMEM((D, WCOLS), jnp.float32),
        pltpu.VMEM((N_IDX + 16,), jnp.int32),
        pltpu.VMEM((SVB, 128), jnp.float32),
        pltpu.VMEM((N_WIN + 21,), jnp.int32),
    ],
    compiler_params=pltpu.CompilerParams(needs_layout_passes=False),
)


def kernel(x, indices, values):
  idx = indices.astype(jnp.int32)
  order = jnp.argsort(idx)
  sidx = idx[order]
  svals = jnp.pad(values[order].astype(jnp.float32), ((0, 0), (0, 96)))
  wstart = jnp.searchsorted(
      sidx, jnp.arange(N_WIN + 5, dtype=jnp.int32) * WCOLS,
      side="left").astype(jnp.int32)

  out_t = _sc_sweep(x.T, sidx, svals, wstart)

  tail_x = x[MAIN_COLS:, :]
  tidx = jnp.where(idx >= MAIN_COLS, idx - MAIN_COLS, N_ROWS - MAIN_COLS)
  tail_out = tail_x.at[tidx].add(values, mode="drop")
  out = out_t.T
  return lax.dynamic_update_slice(out, tail_out, (MAIN_COLS, 0))


# speedup vs baseline: 7.1535x; 1.3451x over previous
import jax
import jax.numpy as jnp
from jax import lax
from jax.experimental import pallas as pl
from jax.experimental.pallas import tpu as pltpu
from jax.experimental.pallas import tpu_sc as plsc

N_ROWS = 1_000_000
D = 32
N_IDX = 16_384
WCOLS = 1536
MAIN_COLS = 999_936
N_WIN = MAIN_COLS // WCOLS
N_WORKERS = 32
VCH = 128
SVB = 136

_mesh = plsc.VectorSubcoreMesh(core_axis_name="c", subcore_axis_name="s")


def _sc_body(xt_hbm, sidx_hbm, sval_hbm, wstart_hbm, out_hbm,
             win_v, sidx_v, sval_v, wstart_v):
  nc = 2
  wid = lax.axis_index("s") * nc + lax.axis_index("c")

  pltpu.sync_copy(sidx_hbm, sidx_v.at[pl.ds(0, N_IDX)])
  pltpu.sync_copy(wstart_hbm, wstart_v.at[pl.ds(0, N_WIN + 5)])

  iota16 = lax.iota(jnp.int32, 16)

  @pl.loop(wid, N_WIN, step=N_WORKERS)
  def _window(w):
    col0 = w * WCOLS
    pltpu.sync_copy(xt_hbm.at[:, pl.ds(col0, WCOLS)], win_v)
    bounds = plsc.load_gather(wstart_v, [w + iota16])
    s0 = bounds[0]
    e0 = bounds[1]
    nch = (e0 - s0 + (VCH - 1)) // VCH

    @pl.loop(0, nch)
    def _chunk(k):
      off = s0 + k * VCH
      offc = jnp.minimum((off // 8) * 8, N_IDX - SVB)
      pltpu.sync_copy(sval_hbm.at[pl.ds(offc, SVB)], sval_v)
      lim = jnp.minimum(e0 - off, VCH)

      @pl.loop(0, lim)
      def _pos(t):
        p = off + t
        rsplat = jnp.full((16,), p - offc, jnp.int32)
        cvec = plsc.load_gather(sidx_v, [jnp.full((16,), p, jnp.int32)])
        cvec = cvec - col0
        v0 = plsc.load_gather(sval_v, [rsplat, iota16])
        v1 = plsc.load_gather(sval_v, [rsplat, iota16 + 16])
        plsc.addupdate_scatter(win_v, [iota16, cvec], v0)
        plsc.addupdate_scatter(win_v, [iota16 + 16, cvec], v1)

    pltpu.sync_copy(win_v, out_hbm.at[:, pl.ds(col0, WCOLS)])


_sc_sweep = pl.kernel(
    _sc_body,
    out_type=jax.ShapeDtypeStruct((D, N_ROWS), jnp.float32),
    mesh=_mesh,
    scratch_types=[
        pltpu.VMEM((D, WCOLS), jnp.float32),
        pltpu.VMEM((N_IDX + 16,), jnp.int32),
        pltpu.VMEM((SVB, 128), jnp.float32),
        pltpu.VMEM((N_WIN + 21,), jnp.int32),
    ],
    compiler_params=pltpu.CompilerParams(needs_layout_passes=False),
)


def kernel(x, indices, values):
  idx = indices.astype(jnp.int32)
  order = jnp.argsort(idx)
  sidx = idx[order]
  svals = jnp.pad(values[order].astype(jnp.float32), ((0, 0), (0, 96)))
  wstart = jnp.searchsorted(
      sidx, jnp.arange(N_WIN + 5, dtype=jnp.int32) * WCOLS,
      side="left", method="compare_all").astype(jnp.int32)

  out_t = _sc_sweep(x.T, sidx, svals, wstart)

  tail_rows = jnp.arange(N_ROWS - MAIN_COLS, dtype=jnp.int32) + MAIN_COLS
  onehot = (tail_rows[:, None] == idx[None, :]).astype(jnp.float32)
  tail_out = x[MAIN_COLS:, :] + onehot @ values
  out = out_t.T
  return lax.dynamic_update_slice(out, tail_out, (MAIN_COLS, 0))
